# baseline (speedup 1.0000x reference)
import functools

import jax
import jax.numpy as jnp
from jax import lax
from jax.experimental import pallas as pl
from jax.experimental.pallas import tpu as pltpu
from jax.experimental.pallas import tpu_sc as plsc

_BETA_PLUS_GAMMA = 1.25


def _norms_body(w_ref, out_ref):
    w = w_ref[...]
    out_ref[...] = jnp.sum(w * w, axis=1)[None, :]


def _norms(W, tk=1024):
    K, D = W.shape
    return pl.pallas_call(
        _norms_body,
        grid=(K // tk,),
        in_specs=[pl.BlockSpec((tk, D), lambda k: (k, 0))],
        out_specs=pl.BlockSpec((1, tk), lambda k: (0, k)),
        out_shape=jax.ShapeDtypeStruct((1, K), jnp.float32),
    )(W)


def _argmin_body(rf_ref, rb_ref, wb_ref, b2_ref, idx_ref, a2_ref, bv_ref,
                 bi_ref, *, nk, tk, use_rsqrt):
    k = pl.program_id(1)

    @pl.when(k == 0)
    def _():
        rf = rf_ref[...]
        a2_ref[...] = jnp.sum(rf * rf, axis=1, keepdims=True)

    s = lax.dot_general(
        rb_ref[...], wb_ref[...], (((1,), (1,)), ((), ())),
        preferred_element_type=jnp.float32)
    t = a2_ref[...] + b2_ref[...]
    d2 = jnp.maximum(t - 2.0 * s, 0.0)
    if use_rsqrt:
        d = d2 * lax.rsqrt(d2)
    else:
        d = jnp.sqrt(d2)
    m = jnp.min(d, axis=1, keepdims=True)
    col = lax.broadcasted_iota(jnp.int32, d.shape, 1)
    li = jnp.min(jnp.where(d <= m, col, jnp.int32(nk * tk)),
                 axis=1, keepdims=True)
    gi = li + k * tk

    @pl.when(k == 0)
    def _():
        bv_ref[...] = m
        bi_ref[...] = gi

    @pl.when(k > 0)
    def _():
        upd = m < bv_ref[...]
        bv_ref[...] = jnp.where(upd, m, bv_ref[...])
        bi_ref[...] = jnp.where(upd, gi, bi_ref[...])

    @pl.when(k == nk - 1)
    def _():
        idx_ref[...] = bi_ref[...]


def _argmin_stage(r, rb, Wb, b2, use_rsqrt, tn=2048, tk=512):
    N, D = r.shape
    K = Wb.shape[0]
    nk = K // tk
    return pl.pallas_call(
        functools.partial(_argmin_body, nk=nk, tk=tk, use_rsqrt=use_rsqrt),
        grid=(N // tn, nk),
        in_specs=[
            pl.BlockSpec((tn, D), lambda n, k: (n, 0)),
            pl.BlockSpec((tn, D), lambda n, k: (n, 0)),
            pl.BlockSpec((tk, D), lambda n, k: (k, 0)),
            pl.BlockSpec((1, tk), lambda n, k: (0, k)),
        ],
        out_specs=pl.BlockSpec((tn, 1), lambda n, k: (n, 0)),
        out_shape=jax.ShapeDtypeStruct((N, 1), jnp.int32),
        scratch_shapes=[pltpu.VMEM((tn, 1), jnp.float32),
                        pltpu.VMEM((tn, 1), jnp.float32),
                        pltpu.VMEM((tn, 1), jnp.int32)],
        compiler_params=pltpu.CompilerParams(
            dimension_semantics=("parallel", "arbitrary")),
    )(r, rb, Wb, b2)


_CHUNK = 128


def _sc_update(r, W, idx, x=None):
    N, D = r.shape
    info = plsc.get_sparse_core_info()
    nc, ns = info.num_cores, info.num_subcores
    nw = nc * ns
    rw = N // nw
    nch = rw // _CHUNK
    final = x is not None
    mesh = plsc.VectorSubcoreMesh(core_axis_name="c", subcore_axis_name="s")

    scratch = [
        pltpu.VMEM((_CHUNK,), jnp.int32),
        pltpu.VMEM((_CHUNK, D), jnp.float32),
        pltpu.VMEM((_CHUNK, D), jnp.float32),
        pltpu.VMEM((16,), jnp.float32),
        pltpu.SemaphoreType.DMA,
    ]
    if final:
        scratch.insert(3, pltpu.VMEM((_CHUNK, D), jnp.float32))

    out_type = (jax.ShapeDtypeStruct((N, D), jnp.float32),
                jax.ShapeDtypeStruct((nw, 16), jnp.float32))

    def body(*refs):
        if final:
            (r_hbm, w_hbm, idx_hbm, x_hbm, out_hbm, lp_hbm,
             idx_v, q_v, r_v, x_v, acc_v, sem) = refs
        else:
            (r_hbm, w_hbm, idx_hbm, out_hbm, lp_hbm,
             idx_v, q_v, r_v, acc_v, sem) = refs
        wid = lax.axis_index("s") * nc + lax.axis_index("c")
        acc_v[...] = jnp.zeros((16,), jnp.float32)

        def chunk(ci, carry):
            base = wid * rw + ci * _CHUNK
            pltpu.sync_copy(idx_hbm.at[pl.ds(base, _CHUNK)], idx_v)
            pltpu.async_copy(w_hbm.at[idx_v], q_v, sem).wait()
            pltpu.sync_copy(r_hbm.at[pl.ds(base, _CHUNK)], r_v)
            if final:
                pltpu.sync_copy(x_hbm.at[pl.ds(base, _CHUNK)], x_v)

            def row(i, acc):
                for j in range(D // 16):
                    sl = pl.ds(j * 16, 16)
                    rv = r_v[i, sl] - q_v[i, sl]
                    if final:
                        x_v[i, sl] = x_v[i, sl] - rv
                    else:
                        r_v[i, sl] = rv
                    acc = acc + rv * rv
                return acc

            acc_v[...] = lax.fori_loop(0, _CHUNK, row, acc_v[...])
            src = x_v if final else r_v
            pltpu.sync_copy(src, out_hbm.at[pl.ds(base, _CHUNK)])
            return carry

        lax.fori_loop(0, nch, chunk, jnp.int32(0))
        pltpu.sync_copy(acc_v, lp_hbm.at[wid])

    fn = pl.kernel(body, out_type=out_type, mesh=mesh, scratch_types=scratch)
    if final:
        return fn(r, W, idx, x)
    return fn(r, W, idx)


def kernel(input, W0, W1, W2):
    N, D = input.shape
    r = input
    out = None
    losses = []
    for si, W in enumerate((W0, W1, W2)):
        b2 = _norms(W)
        idx = _argmin_stage(r, r.astype(jnp.bfloat16),
                            W.astype(jnp.bfloat16), b2,
                            use_rsqrt=False).reshape(-1)
        if si < 2:
            r, lp = _sc_update(r, W, idx)
        else:
            out, lp = _sc_update(r, W, idx, x=input)
        losses.append(_BETA_PLUS_GAMMA * jnp.sum(lp) / (N * D))
    return (out, jnp.stack(losses))


# speedup vs baseline: 1.2970x; 1.2970x over previous
import functools

import jax
import jax.numpy as jnp
from jax import lax
from jax.experimental import pallas as pl
from jax.experimental.pallas import tpu as pltpu
from jax.experimental.pallas import tpu_sc as plsc

_BETA_PLUS_GAMMA = 1.25


def _norms_body(w_ref, out_ref):
    w = w_ref[...]
    out_ref[...] = jnp.sum(w * w, axis=1)[None, :]


def _norms(W, tk=1024):
    K, D = W.shape
    return pl.pallas_call(
        _norms_body,
        grid=(K // tk,),
        in_specs=[pl.BlockSpec((tk, D), lambda k: (k, 0))],
        out_specs=pl.BlockSpec((1, tk), lambda k: (0, k)),
        out_shape=jax.ShapeDtypeStruct((1, K), jnp.float32),
    )(W)


def _argmin_body(rf_ref, rb_ref, wb_ref, b2_ref, idx_ref, a2_ref, bv_ref,
                 bi_ref, *, nk, tk):
    k = pl.program_id(1)

    @pl.when(k == 0)
    def _():
        rf = rf_ref[...]
        a2_ref[...] = jnp.sum(rf * rf, axis=1, keepdims=True)

    s = lax.dot_general(
        rb_ref[...], wb_ref[...], (((1,), (1,)), ((), ())),
        preferred_element_type=jnp.float32)
    t = a2_ref[...] + b2_ref[...]
    d = jnp.maximum(t - 2.0 * s, 0.0)
    m = jnp.min(d, axis=1, keepdims=True)
    col = lax.broadcasted_iota(jnp.int32, d.shape, 1)
    li = jnp.min(jnp.where(d <= m, col, jnp.int32(nk * tk)),
                 axis=1, keepdims=True)
    gi = li + k * tk

    @pl.when(k == 0)
    def _():
        bv_ref[...] = m
        bi_ref[...] = gi

    @pl.when(k > 0)
    def _():
        upd = m < bv_ref[...]
        bv_ref[...] = jnp.where(upd, m, bv_ref[...])
        bi_ref[...] = jnp.where(upd, gi, bi_ref[...])

    @pl.when(k == nk - 1)
    def _():
        idx_ref[...] = bi_ref[...]


def _argmin_stage(r, rb, Wb, b2, tn=2048, tk=512):
    N, D = r.shape
    K = Wb.shape[0]
    nk = K // tk
    return pl.pallas_call(
        functools.partial(_argmin_body, nk=nk, tk=tk),
        grid=(N // tn, nk),
        in_specs=[
            pl.BlockSpec((tn, D), lambda n, k: (n, 0)),
            pl.BlockSpec((tn, D), lambda n, k: (n, 0)),
            pl.BlockSpec((tk, D), lambda n, k: (k, 0)),
            pl.BlockSpec((1, tk), lambda n, k: (0, k)),
        ],
        out_specs=pl.BlockSpec((tn, 1), lambda n, k: (n, 0)),
        out_shape=jax.ShapeDtypeStruct((N, 1), jnp.int32),
        scratch_shapes=[pltpu.VMEM((tn, 1), jnp.float32),
                        pltpu.VMEM((tn, 1), jnp.float32),
                        pltpu.VMEM((tn, 1), jnp.int32)],
        compiler_params=pltpu.CompilerParams(
            dimension_semantics=("parallel", "arbitrary")),
    )(r, rb, Wb, b2)


_CHUNK = 128


def _sc_update(r, W, idx, x=None):
    N, D = r.shape
    info = plsc.get_sparse_core_info()
    nc, ns = info.num_cores, info.num_subcores
    nw = nc * ns
    rw = N // nw
    nch = rw // _CHUNK
    final = x is not None
    mesh = plsc.VectorSubcoreMesh(core_axis_name="c", subcore_axis_name="s")

    scratch = [
        pltpu.VMEM((_CHUNK,), jnp.int32),
        pltpu.VMEM((_CHUNK, D), jnp.float32),
        pltpu.VMEM((_CHUNK, D), jnp.float32),
        pltpu.VMEM((16,), jnp.float32),
        pltpu.SemaphoreType.DMA,
    ]
    if final:
        scratch.insert(3, pltpu.VMEM((_CHUNK, D), jnp.float32))

    out_type = (jax.ShapeDtypeStruct((N, D), jnp.float32),
                jax.ShapeDtypeStruct((nw, 16), jnp.float32))

    def body(*refs):
        if final:
            (r_hbm, w_hbm, idx_hbm, x_hbm, out_hbm, lp_hbm,
             idx_v, q_v, r_v, x_v, acc_v, sem) = refs
        else:
            (r_hbm, w_hbm, idx_hbm, out_hbm, lp_hbm,
             idx_v, q_v, r_v, acc_v, sem) = refs
        wid = lax.axis_index("s") * nc + lax.axis_index("c")
        acc_v[...] = jnp.zeros((16,), jnp.float32)

        def chunk(ci, carry):
            base = wid * rw + ci * _CHUNK
            pltpu.sync_copy(idx_hbm.at[pl.ds(base, _CHUNK)], idx_v)
            pltpu.async_copy(w_hbm.at[idx_v], q_v, sem).wait()
            pltpu.sync_copy(r_hbm.at[pl.ds(base, _CHUNK)], r_v)
            if final:
                pltpu.sync_copy(x_hbm.at[pl.ds(base, _CHUNK)], x_v)

            def row(i, acc):
                for j in range(D // 16):
                    sl = pl.ds(j * 16, 16)
                    rv = r_v[i, sl] - q_v[i, sl]
                    if final:
                        x_v[i, sl] = x_v[i, sl] - rv
                    else:
                        r_v[i, sl] = rv
                    acc = acc + rv * rv
                return acc

            acc_v[...] = lax.fori_loop(0, _CHUNK, row, acc_v[...])
            src = x_v if final else r_v
            pltpu.sync_copy(src, out_hbm.at[pl.ds(base, _CHUNK)])
            return carry

        lax.fori_loop(0, nch, chunk, jnp.int32(0))
        pltpu.sync_copy(acc_v, lp_hbm.at[wid])

    fn = pl.kernel(body, out_type=out_type, mesh=mesh, scratch_types=scratch)
    if final:
        return fn(r, W, idx, x)
    return fn(r, W, idx)


def kernel(input, W0, W1, W2):
    N, D = input.shape
    r = input
    out = None
    losses = []
    for si, W in enumerate((W0, W1, W2)):
        b2 = _norms(W)
        idx = _argmin_stage(r, r.astype(jnp.bfloat16),
                            W.astype(jnp.bfloat16), b2).reshape(-1)
        if si < 2:
            r, lp = _sc_update(r, W, idx)
        else:
            out, lp = _sc_update(r, W, idx, x=input)
        losses.append(_BETA_PLUS_GAMMA * jnp.sum(lp) / (N * D))
    return (out, jnp.stack(losses))


# speedup vs baseline: 1.6640x; 1.2830x over previous
import functools

import jax
import jax.numpy as jnp
from jax import lax
from jax.experimental import pallas as pl
from jax.experimental.pallas import tpu as pltpu
from jax.experimental.pallas import tpu_sc as plsc

_BETA_PLUS_GAMMA = 1.25


def _norms_body(w_ref, out_ref):
    w = w_ref[...]
    out_ref[...] = jnp.sum(w * w, axis=1)[None, :]


def _norms(W, tk=1024):
    K, D = W.shape
    return pl.pallas_call(
        _norms_body,
        grid=(K // tk,),
        in_specs=[pl.BlockSpec((tk, D), lambda k: (k, 0))],
        out_specs=pl.BlockSpec((1, tk), lambda k: (0, k)),
        out_shape=jax.ShapeDtypeStruct((1, K), jnp.float32),
    )(W)


def _argmin_body(rf_ref, rb_ref, wb_ref, b2_ref, idx_ref, a2_ref, bv_ref,
                 bi_ref, *, nk, tk):
    k = pl.program_id(1)

    @pl.when(k == 0)
    def _():
        rf = rf_ref[...]
        a2_ref[...] = jnp.sum(rf * rf, axis=1, keepdims=True)

    s = lax.dot_general(
        rb_ref[...], wb_ref[...], (((1,), (1,)), ((), ())),
        preferred_element_type=jnp.float32)
    t = a2_ref[...] + b2_ref[...]
    d = jnp.maximum(t - 2.0 * s, 0.0)
    m = jnp.min(d, axis=1, keepdims=True)
    col = lax.broadcasted_iota(jnp.int32, d.shape, 1)
    li = jnp.min(jnp.where(d <= m, col, jnp.int32(nk * tk)),
                 axis=1, keepdims=True)
    gi = li + k * tk

    @pl.when(k == 0)
    def _():
        bv_ref[...] = m
        bi_ref[...] = gi

    @pl.when(k > 0)
    def _():
        upd = m < bv_ref[...]
        bv_ref[...] = jnp.where(upd, m, bv_ref[...])
        bi_ref[...] = jnp.where(upd, gi, bi_ref[...])

    @pl.when(k == nk - 1)
    def _():
        idx_ref[...] = bi_ref[...]


def _argmin_stage(r, rb, Wb, b2, tn=2048, tk=1024):
    N, D = r.shape
    K = Wb.shape[0]
    nk = K // tk
    return pl.pallas_call(
        functools.partial(_argmin_body, nk=nk, tk=tk),
        grid=(N // tn, nk),
        in_specs=[
            pl.BlockSpec((tn, D), lambda n, k: (n, 0)),
            pl.BlockSpec((tn, D), lambda n, k: (n, 0)),
            pl.BlockSpec((tk, D), lambda n, k: (k, 0)),
            pl.BlockSpec((1, tk), lambda n, k: (0, k)),
        ],
        out_specs=pl.BlockSpec((tn, 1), lambda n, k: (n, 0)),
        out_shape=jax.ShapeDtypeStruct((N, 1), jnp.int32),
        scratch_shapes=[pltpu.VMEM((tn, 1), jnp.float32),
                        pltpu.VMEM((tn, 1), jnp.float32),
                        pltpu.VMEM((tn, 1), jnp.int32)],
        compiler_params=pltpu.CompilerParams(
            dimension_semantics=("parallel", "arbitrary")),
    )(r, rb, Wb, b2)


_CHUNK = 128


def _sc_update(r, W, idx, x=None):
    N, D = r.shape
    info = plsc.get_sparse_core_info()
    nc, ns = info.num_cores, info.num_subcores
    nw = nc * ns
    rw = N // nw
    nch = rw // _CHUNK
    final = x is not None
    mesh = plsc.VectorSubcoreMesh(core_axis_name="c", subcore_axis_name="s")

    scratch = [
        pltpu.VMEM((_CHUNK,), jnp.int32),
        pltpu.VMEM((_CHUNK, D), jnp.float32),
        pltpu.VMEM((_CHUNK, D), jnp.float32),
        pltpu.VMEM((16,), jnp.float32),
        pltpu.SemaphoreType.DMA,
    ]
    if final:
        scratch.insert(3, pltpu.VMEM((_CHUNK, D), jnp.float32))

    out_type = (jax.ShapeDtypeStruct((N, D), jnp.float32),
                jax.ShapeDtypeStruct((nw, 16), jnp.float32))

    def body(*refs):
        if final:
            (r_hbm, w_hbm, idx_hbm, x_hbm, out_hbm, lp_hbm,
             idx_v, q_v, r_v, x_v, acc_v, sem) = refs
        else:
            (r_hbm, w_hbm, idx_hbm, out_hbm, lp_hbm,
             idx_v, q_v, r_v, acc_v, sem) = refs
        wid = lax.axis_index("s") * nc + lax.axis_index("c")
        acc_v[...] = jnp.zeros((16,), jnp.float32)

        def chunk(ci, carry):
            base = wid * rw + ci * _CHUNK
            pltpu.sync_copy(idx_hbm.at[pl.ds(base, _CHUNK)], idx_v)
            pltpu.async_copy(w_hbm.at[idx_v], q_v, sem).wait()
            pltpu.sync_copy(r_hbm.at[pl.ds(base, _CHUNK)], r_v)
            if final:
                pltpu.sync_copy(x_hbm.at[pl.ds(base, _CHUNK)], x_v)

            def row(i, acc):
                for j in range(D // 16):
                    sl = pl.ds(j * 16, 16)
                    rv = r_v[i, sl] - q_v[i, sl]
                    if final:
                        x_v[i, sl] = x_v[i, sl] - rv
                    else:
                        r_v[i, sl] = rv
                    acc = acc + rv * rv
                return acc

            acc_v[...] = lax.fori_loop(0, _CHUNK, row, acc_v[...])
            src = x_v if final else r_v
            pltpu.sync_copy(src, out_hbm.at[pl.ds(base, _CHUNK)])
            return carry

        lax.fori_loop(0, nch, chunk, jnp.int32(0))
        pltpu.sync_copy(acc_v, lp_hbm.at[wid])

    fn = pl.kernel(body, out_type=out_type, mesh=mesh, scratch_types=scratch)
    if final:
        return fn(r, W, idx, x)
    return fn(r, W, idx)


def kernel(input, W0, W1, W2):
    N, D = input.shape
    r = input
    out = None
    losses = []
    for si, W in enumerate((W0, W1, W2)):
        b2 = _norms(W)
        idx = _argmin_stage(r, r.astype(jnp.bfloat16),
                            W.astype(jnp.bfloat16), b2).reshape(-1)
        if si < 2:
            r, lp = _sc_update(r, W, idx)
        else:
            out, lp = _sc_update(r, W, idx, x=input)
        losses.append(_BETA_PLUS_GAMMA * jnp.sum(lp) / (N * D))
    return (out, jnp.stack(losses))


# speedup vs baseline: 1.8177x; 1.0924x over previous
import functools

import jax
import jax.numpy as jnp
from jax import lax
from jax.experimental import pallas as pl
from jax.experimental.pallas import tpu as pltpu
from jax.experimental.pallas import tpu_sc as plsc

_BETA_PLUS_GAMMA = 1.25


def _norms_body(w_ref, out_ref):
    w = w_ref[...]
    out_ref[...] = jnp.sum(w * w, axis=1)[None, :]


def _norms(W, tk=1024):
    K, D = W.shape
    return pl.pallas_call(
        _norms_body,
        grid=(K // tk,),
        in_specs=[pl.BlockSpec((tk, D), lambda k: (k, 0))],
        out_specs=pl.BlockSpec((1, tk), lambda k: (0, k)),
        out_shape=jax.ShapeDtypeStruct((1, K), jnp.float32),
    )(W)


def _argmin_body(rf_ref, rb_ref, wb_ref, b2_ref, idx_ref, a2_ref, bv_ref,
                 bi_ref, *, nk, tk):
    k = pl.program_id(1)

    @pl.when(k == 0)
    def _():
        rf = rf_ref[...]
        a2_ref[...] = jnp.sum(rf * rf, axis=1, keepdims=True)

    s = lax.dot_general(
        rb_ref[...], wb_ref[...], (((1,), (1,)), ((), ())),
        preferred_element_type=jnp.float32)
    t = a2_ref[...] + b2_ref[...]
    d = jnp.maximum(t - 2.0 * s, 0.0)
    m = jnp.min(d, axis=1, keepdims=True)
    col = lax.broadcasted_iota(jnp.int32, d.shape, 1)
    li = jnp.min(jnp.where(d <= m, col, jnp.int32(nk * tk)),
                 axis=1, keepdims=True)
    gi = li + k * tk

    @pl.when(k == 0)
    def _():
        bv_ref[...] = m
        bi_ref[...] = gi

    @pl.when(k > 0)
    def _():
        upd = m < bv_ref[...]
        bv_ref[...] = jnp.where(upd, m, bv_ref[...])
        bi_ref[...] = jnp.where(upd, gi, bi_ref[...])

    @pl.when(k == nk - 1)
    def _():
        idx_ref[...] = bi_ref[...]


def _argmin_stage(r, rb, Wb, b2, tn=2048, tk=2048):
    N, D = r.shape
    K = Wb.shape[0]
    nk = K // tk
    return pl.pallas_call(
        functools.partial(_argmin_body, nk=nk, tk=tk),
        grid=(N // tn, nk),
        in_specs=[
            pl.BlockSpec((tn, D), lambda n, k: (n, 0)),
            pl.BlockSpec((tn, D), lambda n, k: (n, 0)),
            pl.BlockSpec((tk, D), lambda n, k: (k, 0)),
            pl.BlockSpec((1, tk), lambda n, k: (0, k)),
        ],
        out_specs=pl.BlockSpec((tn, 1), lambda n, k: (n, 0)),
        out_shape=jax.ShapeDtypeStruct((N, 1), jnp.int32),
        scratch_shapes=[pltpu.VMEM((tn, 1), jnp.float32),
                        pltpu.VMEM((tn, 1), jnp.float32),
                        pltpu.VMEM((tn, 1), jnp.int32)],
        compiler_params=pltpu.CompilerParams(
            dimension_semantics=("parallel", "arbitrary")),
    )(r, rb, Wb, b2)


_CHUNK = 128


def _sc_update(r, W, idx, x=None):
    N, D = r.shape
    info = plsc.get_sparse_core_info()
    nc, ns = info.num_cores, info.num_subcores
    nw = nc * ns
    rw = N // nw
    nch = rw // _CHUNK
    final = x is not None
    mesh = plsc.VectorSubcoreMesh(core_axis_name="c", subcore_axis_name="s")

    scratch = [
        pltpu.VMEM((_CHUNK,), jnp.int32),
        pltpu.VMEM((_CHUNK, D), jnp.float32),
        pltpu.VMEM((_CHUNK, D), jnp.float32),
        pltpu.VMEM((16,), jnp.float32),
        pltpu.SemaphoreType.DMA,
    ]
    if final:
        scratch.insert(3, pltpu.VMEM((_CHUNK, D), jnp.float32))

    out_type = (jax.ShapeDtypeStruct((N, D), jnp.float32),
                jax.ShapeDtypeStruct((nw, 16), jnp.float32))

    def body(*refs):
        if final:
            (r_hbm, w_hbm, idx_hbm, x_hbm, out_hbm, lp_hbm,
             idx_v, q_v, r_v, x_v, acc_v, sem) = refs
        else:
            (r_hbm, w_hbm, idx_hbm, out_hbm, lp_hbm,
             idx_v, q_v, r_v, acc_v, sem) = refs
        wid = lax.axis_index("s") * nc + lax.axis_index("c")
        acc_v[...] = jnp.zeros((16,), jnp.float32)

        def chunk(ci, carry):
            base = wid * rw + ci * _CHUNK
            pltpu.sync_copy(idx_hbm.at[pl.ds(base, _CHUNK)], idx_v)
            pltpu.async_copy(w_hbm.at[idx_v], q_v, sem).wait()
            pltpu.sync_copy(r_hbm.at[pl.ds(base, _CHUNK)], r_v)
            if final:
                pltpu.sync_copy(x_hbm.at[pl.ds(base, _CHUNK)], x_v)

            def row(i, acc):
                for j in range(D // 16):
                    sl = pl.ds(j * 16, 16)
                    rv = r_v[i, sl] - q_v[i, sl]
                    if final:
                        x_v[i, sl] = x_v[i, sl] - rv
                    else:
                        r_v[i, sl] = rv
                    acc = acc + rv * rv
                return acc

            acc_v[...] = lax.fori_loop(0, _CHUNK, row, acc_v[...])
            src = x_v if final else r_v
            pltpu.sync_copy(src, out_hbm.at[pl.ds(base, _CHUNK)])
            return carry

        lax.fori_loop(0, nch, chunk, jnp.int32(0))
        pltpu.sync_copy(acc_v, lp_hbm.at[wid])

    fn = pl.kernel(body, out_type=out_type, mesh=mesh, scratch_types=scratch)
    if final:
        return fn(r, W, idx, x)
    return fn(r, W, idx)


def kernel(input, W0, W1, W2):
    N, D = input.shape
    r = input
    out = None
    losses = []
    for si, W in enumerate((W0, W1, W2)):
        b2 = _norms(W)
        idx = _argmin_stage(r, r.astype(jnp.bfloat16),
                            W.astype(jnp.bfloat16), b2).reshape(-1)
        if si < 2:
            r, lp = _sc_update(r, W, idx)
        else:
            out, lp = _sc_update(r, W, idx, x=input)
        losses.append(_BETA_PLUS_GAMMA * jnp.sum(lp) / (N * D))
    return (out, jnp.stack(losses))


# speedup vs baseline: 1.8894x; 1.0394x over previous
import functools

import jax
import jax.numpy as jnp
from jax import lax
from jax.experimental import pallas as pl
from jax.experimental.pallas import tpu as pltpu
from jax.experimental.pallas import tpu_sc as plsc

_BETA_PLUS_GAMMA = 1.25


def _norms_body(w_ref, out_ref):
    w = w_ref[...]
    out_ref[...] = jnp.sum(w * w, axis=1)[None, :]


def _norms(W, tk=1024):
    K, D = W.shape
    return pl.pallas_call(
        _norms_body,
        grid=(K // tk,),
        in_specs=[pl.BlockSpec((tk, D), lambda k: (k, 0))],
        out_specs=pl.BlockSpec((1, tk), lambda k: (0, k)),
        out_shape=jax.ShapeDtypeStruct((1, K), jnp.float32),
    )(W)


def _argmin_body(rf_ref, rb_ref, wb_ref, b2_ref, idx_ref, a2_ref, bv_ref,
                 bi_ref, *, nk, tk):
    k = pl.program_id(1)

    @pl.when(k == 0)
    def _():
        rf = rf_ref[...]
        a2_ref[...] = jnp.sum(rf * rf, axis=1, keepdims=True)

    s = lax.dot_general(
        rb_ref[...], wb_ref[...], (((1,), (1,)), ((), ())),
        preferred_element_type=jnp.float32)
    t = a2_ref[...] + b2_ref[...]
    d = jnp.maximum(t - 2.0 * s, 0.0)
    m = jnp.min(d, axis=1, keepdims=True)
    col = lax.broadcasted_iota(jnp.int32, d.shape, 1)
    li = jnp.min(jnp.where(d <= m, col, jnp.int32(nk * tk)),
                 axis=1, keepdims=True)
    gi = li + k * tk

    @pl.when(k == 0)
    def _():
        bv_ref[...] = m
        bi_ref[...] = gi

    @pl.when(k > 0)
    def _():
        upd = m < bv_ref[...]
        bv_ref[...] = jnp.where(upd, m, bv_ref[...])
        bi_ref[...] = jnp.where(upd, gi, bi_ref[...])

    @pl.when(k == nk - 1)
    def _():
        idx_ref[...] = bi_ref[...]


def _argmin_stage(r, rb, Wb, b2, tn=2048, tk=4096):
    N, D = r.shape
    K = Wb.shape[0]
    nk = K // tk
    return pl.pallas_call(
        functools.partial(_argmin_body, nk=nk, tk=tk),
        grid=(N // tn, nk),
        in_specs=[
            pl.BlockSpec((tn, D), lambda n, k: (n, 0)),
            pl.BlockSpec((tn, D), lambda n, k: (n, 0)),
            pl.BlockSpec((tk, D), lambda n, k: (k, 0)),
            pl.BlockSpec((1, tk), lambda n, k: (0, k)),
        ],
        out_specs=pl.BlockSpec((tn, 1), lambda n, k: (n, 0)),
        out_shape=jax.ShapeDtypeStruct((N, 1), jnp.int32),
        scratch_shapes=[pltpu.VMEM((tn, 1), jnp.float32),
                        pltpu.VMEM((tn, 1), jnp.float32),
                        pltpu.VMEM((tn, 1), jnp.int32)],
        compiler_params=pltpu.CompilerParams(
            dimension_semantics=("parallel", "arbitrary")),
    )(r, rb, Wb, b2)


_CHUNK = 128


def _sc_update(r, W, idx, x=None):
    N, D = r.shape
    info = plsc.get_sparse_core_info()
    nc, ns = info.num_cores, info.num_subcores
    nw = nc * ns
    rw = N // nw
    nch = rw // _CHUNK
    final = x is not None
    mesh = plsc.VectorSubcoreMesh(core_axis_name="c", subcore_axis_name="s")

    scratch = [
        pltpu.VMEM((_CHUNK,), jnp.int32),
        pltpu.VMEM((_CHUNK, D), jnp.float32),
        pltpu.VMEM((_CHUNK, D), jnp.float32),
        pltpu.VMEM((16,), jnp.float32),
        pltpu.SemaphoreType.DMA,
    ]
    if final:
        scratch.insert(3, pltpu.VMEM((_CHUNK, D), jnp.float32))

    out_type = (jax.ShapeDtypeStruct((N, D), jnp.float32),
                jax.ShapeDtypeStruct((nw, 16), jnp.float32))

    def body(*refs):
        if final:
            (r_hbm, w_hbm, idx_hbm, x_hbm, out_hbm, lp_hbm,
             idx_v, q_v, r_v, x_v, acc_v, sem) = refs
        else:
            (r_hbm, w_hbm, idx_hbm, out_hbm, lp_hbm,
             idx_v, q_v, r_v, acc_v, sem) = refs
        wid = lax.axis_index("s") * nc + lax.axis_index("c")
        acc_v[...] = jnp.zeros((16,), jnp.float32)

        def chunk(ci, carry):
            base = wid * rw + ci * _CHUNK
            pltpu.sync_copy(idx_hbm.at[pl.ds(base, _CHUNK)], idx_v)
            pltpu.async_copy(w_hbm.at[idx_v], q_v, sem).wait()
            pltpu.sync_copy(r_hbm.at[pl.ds(base, _CHUNK)], r_v)
            if final:
                pltpu.sync_copy(x_hbm.at[pl.ds(base, _CHUNK)], x_v)

            def row(i, acc):
                for j in range(D // 16):
                    sl = pl.ds(j * 16, 16)
                    rv = r_v[i, sl] - q_v[i, sl]
                    if final:
                        x_v[i, sl] = x_v[i, sl] - rv
                    else:
                        r_v[i, sl] = rv
                    acc = acc + rv * rv
                return acc

            acc_v[...] = lax.fori_loop(0, _CHUNK, row, acc_v[...])
            src = x_v if final else r_v
            pltpu.sync_copy(src, out_hbm.at[pl.ds(base, _CHUNK)])
            return carry

        lax.fori_loop(0, nch, chunk, jnp.int32(0))
        pltpu.sync_copy(acc_v, lp_hbm.at[wid])

    fn = pl.kernel(body, out_type=out_type, mesh=mesh, scratch_types=scratch)
    if final:
        return fn(r, W, idx, x)
    return fn(r, W, idx)


def kernel(input, W0, W1, W2):
    N, D = input.shape
    r = input
    out = None
    losses = []
    for si, W in enumerate((W0, W1, W2)):
        b2 = _norms(W)
        idx = _argmin_stage(r, r.astype(jnp.bfloat16),
                            W.astype(jnp.bfloat16), b2).reshape(-1)
        if si < 2:
            r, lp = _sc_update(r, W, idx)
        else:
            out, lp = _sc_update(r, W, idx, x=input)
        losses.append(_BETA_PLUS_GAMMA * jnp.sum(lp) / (N * D))
    return (out, jnp.stack(losses))
